# baseline (speedup 1.0000x reference)
import jax
import jax.numpy as jnp
from jax.experimental import pallas as pl
from jax.experimental.pallas import tpu as pltpu

_B, _D, _E, _H, _C = 2048, 1024, 16, 256, 100
_CP = 128
_NC = 4
_TC = _B // _NC


def _fused_kernel(x_ref, proto_ref, W1_ref, b1_ref, g_ref, bb_ref,
                  W2_ref, b2_ref, out_ref, w_scratch, x16_scratch):
    e = pl.program_id(0)

    @pl.when(e == 0)
    def _compute_router():
        xf = x_ref[...]
        xn = xf / (jnp.sqrt(jnp.sum(xf * xf, axis=1, keepdims=True)) + 1e-8)
        p = proto_ref[...]
        pn = p / (jnp.sqrt(jnp.sum(p * p, axis=1, keepdims=True)) + 1e-8)
        sims = jnp.dot(xn, pn.T, preferred_element_type=jnp.float32)
        w_scratch[...] = jax.nn.softmax(sims, axis=-1)
        x16_scratch[...] = xf.astype(jnp.bfloat16)

    w1 = W1_ref[0].astype(jnp.bfloat16)
    w2 = W2_ref[0].astype(jnp.bfloat16)
    b1 = b1_ref[0]
    g = g_ref[0]
    bb = bb_ref[0]
    b2 = b2_ref[0]
    w = w_scratch[...]
    lane = jax.lax.broadcasted_iota(jnp.int32, w.shape, 1)
    w_col = jnp.sum(jnp.where(lane == e, w, 0.0), axis=1, keepdims=True)

    for c in range(_NC):
        rows = slice(c * _TC, (c + 1) * _TC)
        xb = x16_scratch[rows, :]
        h = jnp.dot(xb, w1, preferred_element_type=jnp.float32) + b1
        mu = jnp.mean(h, axis=-1, keepdims=True)
        var = jnp.mean(h * h, axis=-1, keepdims=True) - mu * mu
        rstd = jax.lax.rsqrt(var + 1e-5)
        hn = h * rstd - mu * rstd
        hg = hn * g + bb
        hgelu = jax.nn.gelu(hg).astype(jnp.bfloat16)
        logits = (jnp.dot(hgelu, w2, preferred_element_type=jnp.float32)
                  + b2)
        acc = w_col[rows, :] * logits

        @pl.when(e == 0)
        def _init():
            out_ref[rows, :] = acc

        @pl.when(e != 0)
        def _acc():
            out_ref[rows, :] += acc


@jax.jit
def kernel(x, prototypes, W1, b1, ln_g, ln_b, W2, b2):
    W2p = jnp.pad(W2, ((0, 0), (0, 0), (0, _CP - _C)))
    b2p = jnp.pad(b2, ((0, 0), (0, _CP - _C)))
    b1r = b1.reshape(_E, 1, _H)
    gr = ln_g.reshape(_E, 1, _H)
    br = ln_b.reshape(_E, 1, _H)
    b2r = b2p.reshape(_E, 1, _CP)
    out = pl.pallas_call(
        _fused_kernel,
        grid=(_E,),
        in_specs=[
            pl.BlockSpec((_B, _D), lambda e: (0, 0)),
            pl.BlockSpec((_E, _D), lambda e: (0, 0)),
            pl.BlockSpec((1, _D, _H), lambda e: (e, 0, 0)),
            pl.BlockSpec((1, 1, _H), lambda e: (e, 0, 0)),
            pl.BlockSpec((1, 1, _H), lambda e: (e, 0, 0)),
            pl.BlockSpec((1, 1, _H), lambda e: (e, 0, 0)),
            pl.BlockSpec((1, _H, _CP), lambda e: (e, 0, 0)),
            pl.BlockSpec((1, 1, _CP), lambda e: (e, 0, 0)),
        ],
        out_specs=pl.BlockSpec((_B, _CP), lambda e: (0, 0)),
        out_shape=jax.ShapeDtypeStruct((_B, _CP), jnp.float32),
        scratch_shapes=[pltpu.VMEM((_B, _E), jnp.float32),
                        pltpu.VMEM((_B, _D), jnp.bfloat16)],
        compiler_params=pltpu.CompilerParams(
            dimension_semantics=("arbitrary",)),
    )(x, prototypes, W1, b1r, gr, br, W2p, b2r)
    return out[:, :_C]


# speedup vs baseline: 1.3043x; 1.3043x over previous
import jax
import jax.numpy as jnp
from jax.experimental import pallas as pl
from jax.experimental.pallas import tpu as pltpu

_B, _D, _E, _H, _C = 2048, 1024, 16, 256, 100
_CP = 128


def _fused_kernel(x_ref, proto_ref, W1_ref, b1_ref, g_ref, bb_ref,
                  W2_ref, b2_ref, out_ref, w_scratch, x16_scratch, h_scratch):
    e = pl.program_id(0)

    @pl.when(e == 0)
    def _compute_router():
        xf = x_ref[...]
        xn = xf / (jnp.sqrt(jnp.sum(xf * xf, axis=1, keepdims=True)) + 1e-8)
        p = proto_ref[...]
        pn = p / (jnp.sqrt(jnp.sum(p * p, axis=1, keepdims=True)) + 1e-8)
        sims = jnp.dot(xn, pn.T, preferred_element_type=jnp.float32)
        w_scratch[...] = jax.nn.softmax(sims, axis=-1)
        x16_scratch[...] = xf.astype(jnp.bfloat16)

    @pl.when(e < _E)
    def _produce():
        w1 = W1_ref[0].astype(jnp.bfloat16)
        xb = x16_scratch[...]
        h = jnp.dot(xb, w1, preferred_element_type=jnp.float32) + b1_ref[0]
        h_scratch[e % 2] = h

    @pl.when(e > 0)
    def _consume():
        ep = e - 1
        h = h_scratch[ep % 2]
        mu = jnp.mean(h, axis=-1, keepdims=True)
        var = jnp.mean(h * h, axis=-1, keepdims=True) - mu * mu
        rstd = jax.lax.rsqrt(var + 1e-5)
        hn = h * rstd - mu * rstd
        hg = hn * g_ref[0] + bb_ref[0]
        hgelu = jax.nn.gelu(hg).astype(jnp.bfloat16)
        w2 = W2_ref[0].astype(jnp.bfloat16)
        logits = (jnp.dot(hgelu, w2, preferred_element_type=jnp.float32)
                  + b2_ref[0])

        w = w_scratch[...]
        lane = jax.lax.broadcasted_iota(jnp.int32, w.shape, 1)
        w_col = jnp.sum(jnp.where(lane == ep, w, 0.0), axis=1, keepdims=True)
        acc = w_col * logits

        @pl.when(e == 1)
        def _init():
            out_ref[...] = acc

        @pl.when(e > 1)
        def _acc():
            out_ref[...] += acc


@jax.jit
def kernel(x, prototypes, W1, b1, ln_g, ln_b, W2, b2):
    W2p = jnp.pad(W2, ((0, 0), (0, 0), (0, _CP - _C)))
    b2p = jnp.pad(b2, ((0, 0), (0, _CP - _C)))
    b1r = b1.reshape(_E, 1, _H)
    gr = ln_g.reshape(_E, 1, _H)
    br = ln_b.reshape(_E, 1, _H)
    b2r = b2p.reshape(_E, 1, _CP)

    def _prod_ix(e):
        i = jnp.minimum(e, _E - 1)
        return (i, 0, 0)

    def _cons_ix(e):
        i = jnp.maximum(e - 1, 0)
        return (i, 0, 0)

    out = pl.pallas_call(
        _fused_kernel,
        grid=(_E + 1,),
        in_specs=[
            pl.BlockSpec((_B, _D), lambda e: (0, 0)),
            pl.BlockSpec((_E, _D), lambda e: (0, 0)),
            pl.BlockSpec((1, _D, _H), _prod_ix),
            pl.BlockSpec((1, 1, _H), _prod_ix),
            pl.BlockSpec((1, 1, _H), _cons_ix),
            pl.BlockSpec((1, 1, _H), _cons_ix),
            pl.BlockSpec((1, _H, _CP), _cons_ix),
            pl.BlockSpec((1, 1, _CP), _cons_ix),
        ],
        out_specs=pl.BlockSpec((_B, _CP), lambda e: (0, 0)),
        out_shape=jax.ShapeDtypeStruct((_B, _CP), jnp.float32),
        scratch_shapes=[pltpu.VMEM((_B, _E), jnp.float32),
                        pltpu.VMEM((_B, _D), jnp.bfloat16),
                        pltpu.VMEM((2, _B, _H), jnp.float32)],
        compiler_params=pltpu.CompilerParams(
            dimension_semantics=("arbitrary",)),
    )(x, prototypes, W1, b1r, gr, br, W2p, b2r)
    return out[:, :_C]
